# baseline (speedup 1.0000x reference)
import jax
import jax.numpy as jnp
from jax.experimental import pallas as pl
from jax.experimental.pallas import tpu as pltpu

_N, _P, _B, _D, _M = 100000, 256, 4096, 128, 32
_BN = 5000
_G = _N // _BN
_BUF = 4


def _outer(meta_hbm, card_hbm, w_ref, b_ref, pools_ref, out_ref, acc_ref):
    acc_ref[...] = jnp.zeros_like(acc_ref)

    def _step(meta_ref, card_ref):
        pre = jnp.dot(card_ref[...], w_ref[...],
                      preferred_element_type=jnp.float32) + b_ref[...]
        transformed = pre * jax.nn.sigmoid(pre)
        acc_ref[...] += jax.lax.dot_general(
            meta_ref[...].astype(jnp.bfloat16), transformed.astype(jnp.bfloat16),
            (((0,), (0,)), ((), ())),
            preferred_element_type=jnp.float32)

    buf = pl.Buffered(buffer_count=_BUF)
    pltpu.emit_pipeline(
        _step,
        grid=(_G,),
        in_specs=[
            pl.BlockSpec((_BN, _P), lambda i: (i, 0), pipeline_mode=buf),
            pl.BlockSpec((_BN, _D), lambda i: (i, 0), pipeline_mode=buf),
        ],
    )(meta_hbm, card_hbm)

    out_ref[...] = jnp.dot(pools_ref[...], acc_ref[...],
                           preferred_element_type=jnp.float32)


def kernel(batch_pools, metapath, card_embeddings, W, b_dense):
    b2 = b_dense.reshape(1, _M)
    return pl.pallas_call(
        _outer,
        in_specs=[
            pl.BlockSpec(memory_space=pl.ANY),
            pl.BlockSpec(memory_space=pl.ANY),
            pl.BlockSpec(memory_space=pltpu.VMEM),
            pl.BlockSpec(memory_space=pltpu.VMEM),
            pl.BlockSpec(memory_space=pltpu.VMEM),
        ],
        out_specs=pl.BlockSpec(memory_space=pltpu.VMEM),
        out_shape=jax.ShapeDtypeStruct((_B, _M), jnp.float32),
        scratch_shapes=[pltpu.VMEM((_P, _M), jnp.float32)],
    )(metapath, card_embeddings, W, b2, batch_pools)
